# baseline (speedup 1.0000x reference)
import functools

import jax
import jax.numpy as jnp
import numpy as np
from jax import lax
from jax.experimental import pallas as pl
from jax.experimental.pallas import tpu as pltpu
from jax.experimental.pallas import tpu_sc as plsc

N = 10000
K = 16
DIM_IN = 256
DIM_OUT = 512

NC = 2
NS = 16
NW = NC * NS

B = 10240
B_PER_W = B // NW
CH = 8
CHK = CH * K
NCHUNKS = B_PER_W // CH


_ILV = plsc.PackFormat.INTERLEAVED


def _gather_pool_body(edge_hbm, feats_hbm, out_hbm,
                      idx0, idx1, rows0, rows1, pool_v, sem0, sem1):
    wid = lax.axis_index("s") * NC + lax.axis_index("c")
    base = wid * B_PER_W

    def start(c, idx_v, rows_v, sem):
        node0 = base + c * CH
        off = jnp.minimum(node0 * K, N * K - CHK)
        pltpu.sync_copy(edge_hbm.at[pl.ds(pl.multiple_of(off, 8), CHK)],
                        idx_v)
        pltpu.async_copy(feats_hbm.at[idx_v], rows_v, sem)

    def wait_gather(idx_v, rows_v, sem):
        pltpu.make_async_copy(feats_hbm.at[idx_v], rows_v, sem).wait()

    def accum_out(c, rows_v):
        node0 = base + c * CH

        def node_body(n, carry2):
            r0 = n * K
            for g in range(DIM_IN // 32):

                def widen(k):
                    w = rows_v[r0 + k, pl.ds(g * 16, 16)]
                    fe = plsc.bitcast(w << 16, jnp.float32)
                    fo = plsc.bitcast(w & jnp.int32(-65536), jnp.float32)
                    return fe, fo

                acc_a, acc_b = widen(0)
                for k in range(1, K):
                    a, bb = widen(k)
                    acc_a = acc_a + a
                    acc_b = acc_b + bb
                pool_v[n, pl.ds(g * 32, 16)] = acc_a
                pool_v[n, pl.ds(g * 32 + 16, 16)] = acc_b
            return carry2

        lax.fori_loop(0, CH, node_body, 0, unroll=False)
        pltpu.sync_copy(pool_v, out_hbm.at[pl.ds(pl.multiple_of(node0, 8), CH)])

    start(0, idx0, rows0, sem0)

    def pair_body(t, carry):
        start(2 * t + 1, idx1, rows1, sem1)
        wait_gather(idx0, rows0, sem0)
        accum_out(2 * t, rows0)
        start(jnp.minimum(2 * t + 2, NCHUNKS - 1), idx0, rows0, sem0)
        wait_gather(idx1, rows1, sem1)
        accum_out(2 * t + 1, rows1)
        return carry

    lax.fori_loop(0, NCHUNKS // 2, pair_body, 0, unroll=False)
    wait_gather(idx0, rows0, sem0)


_gather_pool = functools.partial(
    pl.kernel,
    out_type=jax.ShapeDtypeStruct((B, DIM_IN), jnp.float32),
    mesh=plsc.VectorSubcoreMesh(
        core_axis_name="c", subcore_axis_name="s", num_cores=NC,
        num_subcores=NS),
    compiler_params=pltpu.CompilerParams(needs_layout_passes=False),
    scratch_types=[
        pltpu.VMEM((CHK,), jnp.int32),
        pltpu.VMEM((CHK,), jnp.int32),
        pltpu.VMEM((CHK, DIM_IN // 2), jnp.int32),
        pltpu.VMEM((CHK, DIM_IN // 2), jnp.int32),
        pltpu.VMEM((CH, DIM_IN), jnp.float32),
        pltpu.SemaphoreType.DMA,
        pltpu.SemaphoreType.DMA,
    ],
)(_gather_pool_body)

_PERM = np.concatenate([
    np.concatenate([np.arange(g * 32, g * 32 + 32, 2),
                    np.arange(g * 32 + 1, g * 32 + 32, 2)])
    for g in range(DIM_IN // 32)
])


def _matmul_body(x_ref, w_ref, b_ref, o_ref):
    acc = jnp.dot(x_ref[...], w_ref[...], preferred_element_type=jnp.float32)
    o_ref[...] = jnp.maximum(acc + b_ref[...], 0.0)


BM = 512


def kernel(feats, edge_dict, W, b):
    edge_flat = edge_dict.astype(jnp.int32).reshape(-1)

    u = lax.bitcast_convert_type(
        feats.astype(jnp.bfloat16), jnp.uint16).astype(jnp.uint32)
    featsw = lax.bitcast_convert_type(
        u[:, 0::2] | (u[:, 1::2] << 16), jnp.int32)
    pooled = _gather_pool(edge_flat, featsw)

    wt = W.T[_PERM, :] * (1.0 / K)
    b2 = b[None, :]

    out = pl.pallas_call(
        _matmul_body,
        grid=(B // BM,),
        in_specs=[
            pl.BlockSpec((BM, DIM_IN), lambda i: (i, 0)),
            pl.BlockSpec((DIM_IN, DIM_OUT), lambda i: (0, 0)),
            pl.BlockSpec((1, DIM_OUT), lambda i: (0, 0)),
        ],
        out_specs=pl.BlockSpec((BM, DIM_OUT), lambda i: (i, 0)),
        out_shape=jax.ShapeDtypeStruct((B, DIM_OUT), jnp.float32),
    )(pooled, wt, b2)

    return out[:N]


# speedup vs baseline: 3.3465x; 3.3465x over previous
import functools

import jax
import jax.numpy as jnp
import numpy as np
from jax import lax
from jax.experimental import pallas as pl
from jax.experimental.pallas import tpu as pltpu
from jax.experimental.pallas import tpu_sc as plsc

N = 10000
K = 16
DIM_IN = 256
DIM_OUT = 512

NC = 2
NS = 16
NW = NC * NS

B = 10240
B_PER_W = B // NW
CH = 8
CHK = CH * K
NCHUNKS = B_PER_W // CH


_ILV = plsc.PackFormat.INTERLEAVED


def _gather_pool_body(edge_hbm, feats_hbm, out_hbm,
                      idx0, idx1, rows0, rows1, pool_v, sem0, sem1):
    wid = lax.axis_index("s") * NC + lax.axis_index("c")
    base = wid * B_PER_W

    def start(c, idx_v, rows_v, sem):
        node0 = base + c * CH
        off = jnp.minimum(node0 * K, N * K - CHK)
        pltpu.sync_copy(edge_hbm.at[pl.ds(pl.multiple_of(off, 8), CHK)],
                        idx_v)
        pltpu.async_copy(feats_hbm.at[idx_v], rows_v, sem)

    def wait_gather(idx_v, rows_v, sem):
        pltpu.make_async_copy(feats_hbm.at[idx_v], rows_v, sem).wait()

    def accum_out(c, rows_v):
        node0 = base + c * CH

        def node_body(n, carry2):
            r0 = n * K
            for g in range(DIM_IN // 32):

                def widen(k):
                    w = rows_v[r0 + k, pl.ds(g * 16, 16)]
                    flo = plsc.bitcast(w << 16, jnp.float32)
                    fhi = plsc.bitcast(w & jnp.int32(-65536), jnp.float32)
                    return flo, fhi

                acc_a, acc_b = widen(0)
                for k in range(1, K):
                    a, bb = widen(k)
                    acc_a = acc_a + a
                    acc_b = acc_b + bb
                pool_v[n, pl.ds(g * 16, 16)] = acc_a
                pool_v[n, pl.ds(DIM_IN // 2 + g * 16, 16)] = acc_b
            return carry2

        lax.fori_loop(0, CH, node_body, 0, unroll=False)
        pltpu.sync_copy(pool_v, out_hbm.at[pl.ds(pl.multiple_of(node0, 8), CH)])

    start(0, idx0, rows0, sem0)

    def pair_body(t, carry):
        start(2 * t + 1, idx1, rows1, sem1)
        wait_gather(idx0, rows0, sem0)
        accum_out(2 * t, rows0)
        start(jnp.minimum(2 * t + 2, NCHUNKS - 1), idx0, rows0, sem0)
        wait_gather(idx1, rows1, sem1)
        accum_out(2 * t + 1, rows1)
        return carry

    lax.fori_loop(0, NCHUNKS // 2, pair_body, 0, unroll=False)
    wait_gather(idx0, rows0, sem0)


_gather_pool = functools.partial(
    pl.kernel,
    out_type=jax.ShapeDtypeStruct((B, DIM_IN), jnp.float32),
    mesh=plsc.VectorSubcoreMesh(
        core_axis_name="c", subcore_axis_name="s", num_cores=NC,
        num_subcores=NS),
    compiler_params=pltpu.CompilerParams(needs_layout_passes=False),
    scratch_types=[
        pltpu.VMEM((CHK,), jnp.int32),
        pltpu.VMEM((CHK,), jnp.int32),
        pltpu.VMEM((CHK, DIM_IN // 2), jnp.int32),
        pltpu.VMEM((CHK, DIM_IN // 2), jnp.int32),
        pltpu.VMEM((CH, DIM_IN), jnp.float32),
        pltpu.SemaphoreType.DMA,
        pltpu.SemaphoreType.DMA,
    ],
)(_gather_pool_body)


def _matmul_body(x_ref, w_ref, b_ref, o_ref):
    acc = jnp.dot(x_ref[...], w_ref[...], preferred_element_type=jnp.float32)
    o_ref[...] = jnp.maximum(acc + b_ref[...], 0.0)


BM = 512


def kernel(feats, edge_dict, W, b):
    edge_flat = edge_dict.astype(jnp.int32).reshape(-1)

    u = lax.bitcast_convert_type(feats.astype(jnp.bfloat16), jnp.uint16)
    lo = u[:, :DIM_IN // 2].astype(jnp.uint32)
    hi = u[:, DIM_IN // 2:].astype(jnp.uint32)
    featsw = lax.bitcast_convert_type(lo | (hi << 16), jnp.int32)
    pooled = _gather_pool(edge_flat, featsw)

    wt = W.T * (1.0 / K)
    b2 = b[None, :]

    out = pl.pallas_call(
        _matmul_body,
        grid=(B // BM,),
        in_specs=[
            pl.BlockSpec((BM, DIM_IN), lambda i: (i, 0)),
            pl.BlockSpec((DIM_IN, DIM_OUT), lambda i: (0, 0)),
            pl.BlockSpec((1, DIM_OUT), lambda i: (0, 0)),
        ],
        out_specs=pl.BlockSpec((BM, DIM_OUT), lambda i: (i, 0)),
        out_shape=jax.ShapeDtypeStruct((B, DIM_OUT), jnp.float32),
    )(pooled, wt, b2)

    return out[:N]
